# Initial kernel scaffold; baseline (speedup 1.0000x reference)
import jax
import jax.numpy as jnp
from jax.experimental import pallas as pl


def kernel(means3D, means2D, opacities, colors_precomp, scales, rotations, backward_mask):
    raise NotImplementedError("write your pallas kernel here")


# speedup vs baseline: 1.6681x; 1.6681x over previous
import functools

import jax
import jax.numpy as jnp
from jax.experimental import pallas as pl

H_IMG = 128
W_IMG = 128
TANX = 0.5
TANY = 0.5
N_GAUSS = 1024
NPARAM = 16

PIX_BLOCK = 256
G_CHUNK = 256

BIG_KEY = 1e30


def _bf(v):
    return v.astype(jnp.bfloat16).astype(jnp.float32)


def _preprocess_kernel(m_ref, op_ref, col_ref, sc_ref, rot_ref,
                       params_ref, radii_ref):
    fx = W_IMG / (2.0 * TANX)
    fy = H_IMG / (2.0 * TANY)

    tx0 = _bf(m_ref[:, 0:1])
    ty0 = _bf(m_ref[:, 1:2])
    tz = _bf(m_ref[:, 2:3]) + 5.0
    valid0 = tz > 0.2
    tzs = jnp.where(valid0, tz, 1.0)

    txtz = jnp.clip(tx0 / tzs, -1.3 * TANX, 1.3 * TANX)
    tytz = jnp.clip(ty0 / tzs, -1.3 * TANY, 1.3 * TANY)
    tx = txtz * tzs
    ty = tytz * tzs

    qr = rot_ref[:, 0:1]; qx = rot_ref[:, 1:2]
    qy = rot_ref[:, 2:3]; qz = rot_ref[:, 3:4]
    qn = jnp.sqrt(qr * qr + qx * qx + qy * qy + qz * qz)
    r = qr / qn; x = qx / qn; y = qy / qn; z = qz / qn
    R00 = 1 - 2 * (y * y + z * z); R01 = 2 * (x * y - r * z); R02 = 2 * (x * z + r * y)
    R10 = 2 * (x * y + r * z); R11 = 1 - 2 * (x * x + z * z); R12 = 2 * (y * z - r * x)
    R20 = 2 * (x * z - r * y); R21 = 2 * (y * z + r * x); R22 = 1 - 2 * (x * x + y * y)

    s0 = sc_ref[:, 0:1]; s1 = sc_ref[:, 1:2]; s2c = sc_ref[:, 2:3]
    v0 = s0 * s0; v1 = s1 * s1; v2 = s2c * s2c
    Rb = [[_bf(R00), _bf(R01), _bf(R02)],
          [_bf(R10), _bf(R11), _bf(R12)],
          [_bf(R20), _bf(R21), _bf(R22)]]
    P = [[_bf(R00 * v0), _bf(R01 * v1), _bf(R02 * v2)],
         [_bf(R10 * v0), _bf(R11 * v1), _bf(R12 * v2)],
         [_bf(R20 * v0), _bf(R21 * v1), _bf(R22 * v2)]]
    S = [[Rb[i][0] * P[k][0] + Rb[i][1] * P[k][1] + Rb[i][2] * P[k][2]
          for k in range(3)] for i in range(3)]

    j00 = _bf(fx / tzs)
    j02 = _bf(-fx * tx / (tzs * tzs))
    j11 = _bf(fy / tzs)
    j12 = _bf(-fy * ty / (tzs * tzs))

    Sb = [[_bf(S[i][k]) for k in range(3)] for i in range(3)]
    t00 = j00 * Sb[0][0] + j02 * Sb[2][0]
    t01 = j00 * Sb[0][1] + j02 * Sb[2][1]
    t02 = j00 * Sb[0][2] + j02 * Sb[2][2]
    t11 = j11 * Sb[1][1] + j12 * Sb[2][1]
    t12 = j11 * Sb[1][2] + j12 * Sb[2][2]
    c00 = _bf(t00) * j00 + _bf(t02) * j02
    c01 = _bf(t01) * j11 + _bf(t02) * j12
    c11 = _bf(t11) * j11 + _bf(t12) * j12

    a = c00 + 0.3
    c = c11 + 0.3
    b = c01
    det = a * c - b * b
    valid = valid0 & (det > 0)
    dets = jnp.where(valid, det, 1.0)
    con_a = c / dets
    con_b = -b / dets
    con_c = a / dets

    mid = 0.5 * (a + c)
    lam1 = mid + jnp.sqrt(jnp.maximum(0.1, mid * mid - det))
    radii = jnp.where(valid, jnp.ceil(3.0 * jnp.sqrt(lam1)), 0.0)
    radii_ref[:, :] = radii.astype(jnp.int32)

    ndc_x = (tx0 / tzs) / TANX
    ndc_y = (ty0 / tzs) / TANY
    px = ((ndc_x + 1.0) * W_IMG - 1.0) * 0.5
    py = ((ndc_y + 1.0) * H_IMG - 1.0) * 0.5

    key = jnp.where(valid, tz, BIG_KEY)
    op_eff = jnp.where(valid, op_ref[:, 0:1], 0.0)

    params_ref[:, 0:1] = key
    params_ref[:, 1:2] = px
    params_ref[:, 2:3] = py
    params_ref[:, 3:4] = con_a
    params_ref[:, 4:5] = con_b
    params_ref[:, 5:6] = con_c
    params_ref[:, 6:7] = op_eff
    params_ref[:, 7:8] = col_ref[:, 0:1]
    params_ref[:, 8:9] = col_ref[:, 1:2]
    params_ref[:, 9:10] = col_ref[:, 2:3]
    params_ref[:, 10:NPARAM] = jnp.zeros((N_GAUSS, NPARAM - 10), jnp.float32)


def _sort_kernel(params_ref, params_t_ref, sorted_t_ref):
    key_col = params_ref[:, 0:1]
    key_row = params_t_ref[0:1, :]
    ii = jax.lax.broadcasted_iota(jnp.int32, (N_GAUSS, N_GAUSS), 0)
    jj = jax.lax.broadcasted_iota(jnp.int32, (N_GAUSS, N_GAUSS), 1)
    lt = (key_row < key_col) | ((key_row == key_col) & (jj < ii))
    rank = jnp.sum(lt.astype(jnp.int32), axis=1, keepdims=True)
    G = (jj == rank).astype(jnp.float32)
    sorted_t_ref[:, :] = jax.lax.dot_general(
        params_t_ref[:, :], G, (((1,), (0,)), ((), ())),
        precision=jax.lax.Precision.HIGHEST,
        preferred_element_type=jnp.float32)


def _raster_kernel(sorted_t_ref, img_ref):
    i = pl.program_id(0)
    p = i * PIX_BLOCK + jax.lax.broadcasted_iota(jnp.int32, (PIX_BLOCK, 1), 0)
    gx = (p % W_IMG).astype(jnp.float32)
    gy = (p // W_IMG).astype(jnp.float32)

    ik = jax.lax.broadcasted_iota(jnp.int32, (G_CHUNK, G_CHUNK), 0)
    ij = jax.lax.broadcasted_iota(jnp.int32, (G_CHUNK, G_CHUNK), 1)
    U = (ik < ij).astype(jnp.float32)

    carry = jnp.zeros((PIX_BLOCK, 1), jnp.float32)
    r_acc = jnp.zeros((PIX_BLOCK, 1), jnp.float32)
    g_acc = jnp.zeros((PIX_BLOCK, 1), jnp.float32)
    b_acc = jnp.zeros((PIX_BLOCK, 1), jnp.float32)

    for cidx in range(N_GAUSS // G_CHUNK):
        sl = slice(cidx * G_CHUNK, (cidx + 1) * G_CHUNK)
        px = sorted_t_ref[1:2, sl]
        py = sorted_t_ref[2:3, sl]
        ca = sorted_t_ref[3:4, sl]
        cb = sorted_t_ref[4:5, sl]
        cc = sorted_t_ref[5:6, sl]
        op = sorted_t_ref[6:7, sl]
        colr = _bf(sorted_t_ref[7:8, sl])
        colg = _bf(sorted_t_ref[8:9, sl])
        colb = _bf(sorted_t_ref[9:10, sl])

        dx = px - gx
        dy = py - gy
        power = -0.5 * (ca * dx * dx + cc * dy * dy) - cb * dx * dy
        al = op * jnp.exp(jnp.minimum(power, 0.0))
        al = jnp.where(power <= 0.0, jnp.minimum(al, 0.99), 0.0)
        al = jnp.where(al >= 1.0 / 255.0, al, 0.0)
        s = jnp.log(1.0 - al)
        excl = jax.lax.dot(s, U, precision=jax.lax.Precision.HIGHEST,
                           preferred_element_type=jnp.float32)
        tprev = jnp.exp(excl + carry)
        w = _bf(al * tprev)
        r_acc = r_acc + jnp.sum(w * colr, axis=1, keepdims=True)
        g_acc = g_acc + jnp.sum(w * colg, axis=1, keepdims=True)
        b_acc = b_acc + jnp.sum(w * colb, axis=1, keepdims=True)
        carry = carry + excl[:, G_CHUNK - 1:G_CHUNK] + s[:, G_CHUNK - 1:G_CHUNK]

    img_ref[:, 0:1] = r_acc
    img_ref[:, 1:2] = g_acc
    img_ref[:, 2:3] = b_acc


@functools.partial(jax.jit, static_argnames=())
def kernel(means3D, means2D, opacities, colors_precomp, scales, rotations,
           backward_mask):
    del means2D, backward_mask
    params, radii = pl.pallas_call(
        _preprocess_kernel,
        out_shape=(
            jax.ShapeDtypeStruct((N_GAUSS, NPARAM), jnp.float32),
            jax.ShapeDtypeStruct((N_GAUSS, 1), jnp.int32),
        ),
    )(means3D, opacities, colors_precomp, scales, rotations)

    params_t = params.T

    sorted_t = pl.pallas_call(
        _sort_kernel,
        out_shape=jax.ShapeDtypeStruct((NPARAM, N_GAUSS), jnp.float32),
    )(params, params_t)

    img = pl.pallas_call(
        _raster_kernel,
        grid=(H_IMG * W_IMG // PIX_BLOCK,),
        in_specs=[pl.BlockSpec((NPARAM, N_GAUSS), lambda i: (0, 0))],
        out_specs=pl.BlockSpec((PIX_BLOCK, 3), lambda i: (i, 0)),
        out_shape=jax.ShapeDtypeStruct((H_IMG * W_IMG, 3), jnp.float32),
    )(sorted_t)

    color = img.reshape(H_IMG, W_IMG, 3).transpose(2, 0, 1)
    return color, radii.reshape(N_GAUSS)


# speedup vs baseline: 2.0516x; 1.2299x over previous
import functools

import jax
import jax.numpy as jnp
from jax.experimental import pallas as pl

H_IMG = 128
W_IMG = 128
TANX = 0.5
TANY = 0.5
N_GAUSS = 1024
NPARAM = 16

PIX_BLOCK = 256
G_CHUNK = 256

BIG_KEY = 1e30


def _bf(v):
    return v.astype(jnp.bfloat16).astype(jnp.float32)


def _preprocess_kernel(m_ref, op_ref, col_ref, sc_ref, rot_ref,
                       params_ref, radii_ref):
    fx = W_IMG / (2.0 * TANX)
    fy = H_IMG / (2.0 * TANY)

    tx0 = _bf(m_ref[:, 0:1])
    ty0 = _bf(m_ref[:, 1:2])
    tz = _bf(m_ref[:, 2:3]) + 5.0
    valid0 = tz > 0.2
    tzs = jnp.where(valid0, tz, 1.0)

    txtz = jnp.clip(tx0 / tzs, -1.3 * TANX, 1.3 * TANX)
    tytz = jnp.clip(ty0 / tzs, -1.3 * TANY, 1.3 * TANY)
    tx = txtz * tzs
    ty = tytz * tzs

    qr = rot_ref[:, 0:1]; qx = rot_ref[:, 1:2]
    qy = rot_ref[:, 2:3]; qz = rot_ref[:, 3:4]
    qn = jnp.sqrt(qr * qr + qx * qx + qy * qy + qz * qz)
    r = qr / qn; x = qx / qn; y = qy / qn; z = qz / qn
    R00 = 1 - 2 * (y * y + z * z); R01 = 2 * (x * y - r * z); R02 = 2 * (x * z + r * y)
    R10 = 2 * (x * y + r * z); R11 = 1 - 2 * (x * x + z * z); R12 = 2 * (y * z - r * x)
    R20 = 2 * (x * z - r * y); R21 = 2 * (y * z + r * x); R22 = 1 - 2 * (x * x + y * y)

    s0 = sc_ref[:, 0:1]; s1 = sc_ref[:, 1:2]; s2c = sc_ref[:, 2:3]
    v0 = s0 * s0; v1 = s1 * s1; v2 = s2c * s2c
    Rb = [[_bf(R00), _bf(R01), _bf(R02)],
          [_bf(R10), _bf(R11), _bf(R12)],
          [_bf(R20), _bf(R21), _bf(R22)]]
    P = [[_bf(R00 * v0), _bf(R01 * v1), _bf(R02 * v2)],
         [_bf(R10 * v0), _bf(R11 * v1), _bf(R12 * v2)],
         [_bf(R20 * v0), _bf(R21 * v1), _bf(R22 * v2)]]
    S = [[Rb[i][0] * P[k][0] + Rb[i][1] * P[k][1] + Rb[i][2] * P[k][2]
          for k in range(3)] for i in range(3)]

    j00 = _bf(fx / tzs)
    j02 = _bf(-fx * tx / (tzs * tzs))
    j11 = _bf(fy / tzs)
    j12 = _bf(-fy * ty / (tzs * tzs))

    Sb = [[_bf(S[i][k]) for k in range(3)] for i in range(3)]
    t00 = j00 * Sb[0][0] + j02 * Sb[2][0]
    t01 = j00 * Sb[0][1] + j02 * Sb[2][1]
    t02 = j00 * Sb[0][2] + j02 * Sb[2][2]
    t11 = j11 * Sb[1][1] + j12 * Sb[2][1]
    t12 = j11 * Sb[1][2] + j12 * Sb[2][2]
    c00 = _bf(t00) * j00 + _bf(t02) * j02
    c01 = _bf(t01) * j11 + _bf(t02) * j12
    c11 = _bf(t11) * j11 + _bf(t12) * j12

    a = c00 + 0.3
    c = c11 + 0.3
    b = c01
    det = a * c - b * b
    valid = valid0 & (det > 0)
    dets = jnp.where(valid, det, 1.0)
    con_a = c / dets
    con_b = -b / dets
    con_c = a / dets

    mid = 0.5 * (a + c)
    lam1 = mid + jnp.sqrt(jnp.maximum(0.1, mid * mid - det))
    radii = jnp.where(valid, jnp.ceil(3.0 * jnp.sqrt(lam1)), 0.0)
    radii_ref[:, :] = radii.astype(jnp.int32)

    ndc_x = (tx0 / tzs) / TANX
    ndc_y = (ty0 / tzs) / TANY
    px = ((ndc_x + 1.0) * W_IMG - 1.0) * 0.5
    py = ((ndc_y + 1.0) * H_IMG - 1.0) * 0.5

    key = jnp.where(valid, tz, BIG_KEY)
    op_eff = jnp.where(valid, op_ref[:, 0:1], 0.0)

    params_ref[:, 0:1] = key
    params_ref[:, 1:2] = px
    params_ref[:, 2:3] = py
    params_ref[:, 3:4] = con_a
    params_ref[:, 4:5] = con_b
    params_ref[:, 5:6] = con_c
    params_ref[:, 6:7] = op_eff
    params_ref[:, 7:8] = col_ref[:, 0:1]
    params_ref[:, 8:9] = col_ref[:, 1:2]
    params_ref[:, 9:10] = col_ref[:, 2:3]
    params_ref[:, 10:NPARAM] = jnp.zeros((N_GAUSS, NPARAM - 10), jnp.float32)


def _sort_kernel(params_ref, params_t_ref, sorted_t_ref):
    key_col = params_ref[:, 0:1]
    key_row = params_t_ref[0:1, :]
    ii = jax.lax.broadcasted_iota(jnp.int32, (N_GAUSS, N_GAUSS), 0)
    jj = jax.lax.broadcasted_iota(jnp.int32, (N_GAUSS, N_GAUSS), 1)
    lt = (key_row < key_col) | ((key_row == key_col) & (jj < ii))
    rank = jnp.sum(lt.astype(jnp.int32), axis=1, keepdims=True)
    G = (jj == rank).astype(jnp.float32)
    sorted_t_ref[:, :] = jax.lax.dot_general(
        params_t_ref[:, :], G, (((1,), (0,)), ((), ())),
        precision=jax.lax.Precision.HIGHEST,
        preferred_element_type=jnp.float32)


def _raster_kernel(sorted_t_ref, img_ref):
    i = pl.program_id(0)
    p = i * PIX_BLOCK + jax.lax.broadcasted_iota(jnp.int32, (PIX_BLOCK, 1), 0)
    gx = (p % W_IMG).astype(jnp.float32)
    gy = (p // W_IMG).astype(jnp.float32)

    ik = jax.lax.broadcasted_iota(jnp.int32, (G_CHUNK, G_CHUNK), 0)
    ij = jax.lax.broadcasted_iota(jnp.int32, (G_CHUNK, G_CHUNK), 1)
    U = (ik < ij).astype(jnp.float32)

    carry = jnp.zeros((PIX_BLOCK, 1), jnp.float32)
    r_acc = jnp.zeros((PIX_BLOCK, 1), jnp.float32)
    g_acc = jnp.zeros((PIX_BLOCK, 1), jnp.float32)
    b_acc = jnp.zeros((PIX_BLOCK, 1), jnp.float32)

    for cidx in range(N_GAUSS // G_CHUNK):
        sl = slice(cidx * G_CHUNK, (cidx + 1) * G_CHUNK)
        px = sorted_t_ref[1:2, sl]
        py = sorted_t_ref[2:3, sl]
        ca = sorted_t_ref[3:4, sl]
        cb = sorted_t_ref[4:5, sl]
        cc = sorted_t_ref[5:6, sl]
        op = sorted_t_ref[6:7, sl]
        colr = _bf(sorted_t_ref[7:8, sl])
        colg = _bf(sorted_t_ref[8:9, sl])
        colb = _bf(sorted_t_ref[9:10, sl])

        dx = px - gx
        dy = py - gy
        power = -0.5 * (ca * dx * dx + cc * dy * dy) - cb * dx * dy
        al = op * jnp.exp(jnp.minimum(power, 0.0))
        al = jnp.where(power <= 0.0, jnp.minimum(al, 0.99), 0.0)
        al = jnp.where(al >= 1.0 / 255.0, al, 0.0)
        s = jnp.log(1.0 - al)
        s_hi = s.astype(jnp.bfloat16)
        s_lo = (s - s_hi.astype(jnp.float32)).astype(jnp.bfloat16)
        excl = (jax.lax.dot(s_hi, U.astype(jnp.bfloat16),
                            preferred_element_type=jnp.float32)
                + jax.lax.dot(s_lo, U.astype(jnp.bfloat16),
                              preferred_element_type=jnp.float32))
        tprev = jnp.exp(excl + carry)
        w = _bf(al * tprev)
        r_acc = r_acc + jnp.sum(w * colr, axis=1, keepdims=True)
        g_acc = g_acc + jnp.sum(w * colg, axis=1, keepdims=True)
        b_acc = b_acc + jnp.sum(w * colb, axis=1, keepdims=True)
        carry = carry + excl[:, G_CHUNK - 1:G_CHUNK] + s[:, G_CHUNK - 1:G_CHUNK]

    img_ref[:, 0:1] = r_acc
    img_ref[:, 1:2] = g_acc
    img_ref[:, 2:3] = b_acc


@functools.partial(jax.jit, static_argnames=())
def kernel(means3D, means2D, opacities, colors_precomp, scales, rotations,
           backward_mask):
    del means2D, backward_mask
    params, radii = pl.pallas_call(
        _preprocess_kernel,
        out_shape=(
            jax.ShapeDtypeStruct((N_GAUSS, NPARAM), jnp.float32),
            jax.ShapeDtypeStruct((N_GAUSS, 1), jnp.int32),
        ),
    )(means3D, opacities, colors_precomp, scales, rotations)

    params_t = params.T

    sorted_t = pl.pallas_call(
        _sort_kernel,
        out_shape=jax.ShapeDtypeStruct((NPARAM, N_GAUSS), jnp.float32),
    )(params, params_t)

    img = pl.pallas_call(
        _raster_kernel,
        grid=(H_IMG * W_IMG // PIX_BLOCK,),
        in_specs=[pl.BlockSpec((NPARAM, N_GAUSS), lambda i: (0, 0))],
        out_specs=pl.BlockSpec((PIX_BLOCK, 3), lambda i: (i, 0)),
        out_shape=jax.ShapeDtypeStruct((H_IMG * W_IMG, 3), jnp.float32),
    )(sorted_t)

    color = img.reshape(H_IMG, W_IMG, 3).transpose(2, 0, 1)
    return color, radii.reshape(N_GAUSS)


# speedup vs baseline: 2.4184x; 1.1788x over previous
import jax
import jax.numpy as jnp
from jax import lax
from jax.experimental import pallas as pl
from jax.experimental.pallas import tpu as pltpu
from jax.experimental.pallas import tpu_sc as plsc

H_IMG = 128
W_IMG = 128
TANX = 0.5
TANY = 0.5
N_GAUSS = 1024
NPARAM = 16

TILE = 16
NTILES = (H_IMG // TILE) * (W_IMG // TILE)
TPIX = TILE * TILE
G_CHUNK = 128

BIG_KEY = 1e30
ALPHA_MIN = 1.0 / 255.0


def _bf(v):
    return v.astype(jnp.bfloat16).astype(jnp.float32)


def _preprocess_kernel(m_ref, op_ref, col_ref, sc_ref, rot_ref,
                       params_ref, radii_ref):
    fx = W_IMG / (2.0 * TANX)
    fy = H_IMG / (2.0 * TANY)

    tx0 = _bf(m_ref[:, 0:1])
    ty0 = _bf(m_ref[:, 1:2])
    tz = _bf(m_ref[:, 2:3]) + 5.0
    valid0 = tz > 0.2
    tzs = jnp.where(valid0, tz, 1.0)

    txtz = jnp.clip(tx0 / tzs, -1.3 * TANX, 1.3 * TANX)
    tytz = jnp.clip(ty0 / tzs, -1.3 * TANY, 1.3 * TANY)
    tx = txtz * tzs
    ty = tytz * tzs

    qr = rot_ref[:, 0:1]; qx = rot_ref[:, 1:2]
    qy = rot_ref[:, 2:3]; qz = rot_ref[:, 3:4]
    qn = jnp.sqrt(qr * qr + qx * qx + qy * qy + qz * qz)
    r = qr / qn; x = qx / qn; y = qy / qn; z = qz / qn
    R00 = 1 - 2 * (y * y + z * z); R01 = 2 * (x * y - r * z); R02 = 2 * (x * z + r * y)
    R10 = 2 * (x * y + r * z); R11 = 1 - 2 * (x * x + z * z); R12 = 2 * (y * z - r * x)
    R20 = 2 * (x * z - r * y); R21 = 2 * (y * z + r * x); R22 = 1 - 2 * (x * x + y * y)

    s0 = sc_ref[:, 0:1]; s1 = sc_ref[:, 1:2]; s2c = sc_ref[:, 2:3]
    v0 = s0 * s0; v1 = s1 * s1; v2 = s2c * s2c
    Rb = [[_bf(R00), _bf(R01), _bf(R02)],
          [_bf(R10), _bf(R11), _bf(R12)],
          [_bf(R20), _bf(R21), _bf(R22)]]
    P = [[_bf(R00 * v0), _bf(R01 * v1), _bf(R02 * v2)],
         [_bf(R10 * v0), _bf(R11 * v1), _bf(R12 * v2)],
         [_bf(R20 * v0), _bf(R21 * v1), _bf(R22 * v2)]]
    S = [[Rb[i][0] * P[k][0] + Rb[i][1] * P[k][1] + Rb[i][2] * P[k][2]
          for k in range(3)] for i in range(3)]

    j00 = _bf(fx / tzs)
    j02 = _bf(-fx * tx / (tzs * tzs))
    j11 = _bf(fy / tzs)
    j12 = _bf(-fy * ty / (tzs * tzs))

    Sb = [[_bf(S[i][k]) for k in range(3)] for i in range(3)]
    t00 = j00 * Sb[0][0] + j02 * Sb[2][0]
    t01 = j00 * Sb[0][1] + j02 * Sb[2][1]
    t02 = j00 * Sb[0][2] + j02 * Sb[2][2]
    t11 = j11 * Sb[1][1] + j12 * Sb[2][1]
    t12 = j11 * Sb[1][2] + j12 * Sb[2][2]
    c00 = _bf(t00) * j00 + _bf(t02) * j02
    c01 = _bf(t01) * j11 + _bf(t02) * j12
    c11 = _bf(t11) * j11 + _bf(t12) * j12

    a = c00 + 0.3
    c = c11 + 0.3
    b = c01
    det = a * c - b * b
    valid = valid0 & (det > 0)
    dets = jnp.where(valid, det, 1.0)
    con_a = c / dets
    con_b = -b / dets
    con_c = a / dets

    mid = 0.5 * (a + c)
    lam1 = mid + jnp.sqrt(jnp.maximum(0.1, mid * mid - det))
    radii = jnp.where(valid, jnp.ceil(3.0 * jnp.sqrt(lam1)), 0.0)
    radii_ref[:, :] = radii.astype(jnp.int32)

    ndc_x = (tx0 / tzs) / TANX
    ndc_y = (ty0 / tzs) / TANY
    px = ((ndc_x + 1.0) * W_IMG - 1.0) * 0.5
    py = ((ndc_y + 1.0) * H_IMG - 1.0) * 0.5

    key = jnp.where(valid, tz, BIG_KEY)
    op_eff = jnp.where(valid, op_ref[:, 0:1], 0.0)

    rr = jnp.maximum(jnp.log(255.0 * jnp.maximum(op_eff, 1e-30)), 0.0)
    visible = op_eff > ALPHA_MIN
    ext_x = jnp.where(visible,
                      jnp.sqrt(jnp.maximum(2.0 * rr * a, 0.0)) + 0.125, -1.0)
    ext_y = jnp.where(visible,
                      jnp.sqrt(jnp.maximum(2.0 * rr * c, 0.0)) + 0.125, -1.0)

    params_ref[:, 0:1] = key
    params_ref[:, 1:2] = px
    params_ref[:, 2:3] = py
    params_ref[:, 3:4] = con_a
    params_ref[:, 4:5] = con_b
    params_ref[:, 5:6] = con_c
    params_ref[:, 6:7] = op_eff
    params_ref[:, 7:8] = col_ref[:, 0:1]
    params_ref[:, 8:9] = col_ref[:, 1:2]
    params_ref[:, 9:10] = col_ref[:, 2:3]
    params_ref[:, 10:11] = ext_x
    params_ref[:, 11:12] = ext_y
    params_ref[:, 12:NPARAM] = jnp.zeros((N_GAUSS, NPARAM - 12), jnp.float32)


def _sort_kernel(params_ref, params_t_ref, sorted_ref, sorted_t_ref):
    key_col = params_ref[:, 0:1]
    key_row = params_t_ref[0:1, :]
    ii = lax.broadcasted_iota(jnp.int32, (N_GAUSS, N_GAUSS), 0)
    jj = lax.broadcasted_iota(jnp.int32, (N_GAUSS, N_GAUSS), 1)
    lt = (key_row < key_col) | ((key_row == key_col) & (jj < ii))
    rank = jnp.sum(lt.astype(jnp.int32), axis=1, keepdims=True)
    G = (jj == rank).astype(jnp.float32)
    sorted_t_ref[:, :] = lax.dot_general(
        params_t_ref[:, :], G, (((1,), (0,)), ((), ())),
        precision=lax.Precision.HIGHEST,
        preferred_element_type=jnp.float32)
    lt2 = (key_col < key_row) | ((key_col == key_row) & (ii < jj))
    rank_row = jnp.sum(lt2.astype(jnp.int32), axis=0, keepdims=True)
    G2 = (ii == rank_row).astype(jnp.float32)
    sorted_ref[:, :] = lax.dot_general(
        G2, params_ref[:, :], (((1,), (0,)), ((), ())),
        precision=lax.Precision.HIGHEST,
        preferred_element_type=jnp.float32)


def _cull_kernel(px_hbm, py_hbm, ex_hbm, ey_hbm, hit_out,
                 px_v, py_v, ex_v, ey_v, hit_v):
    wid = lax.axis_index("s") * 2 + lax.axis_index("c")
    pltpu.sync_copy(px_hbm, px_v)
    pltpu.sync_copy(py_hbm, py_v)
    pltpu.sync_copy(ex_hbm, ex_v)
    pltpu.sync_copy(ey_hbm, ey_v)
    one = jnp.ones((16,), jnp.int32)
    zero = jnp.zeros((16,), jnp.int32)
    for tloc in range(2):
        t = wid * 2 + tloc
        txmin = lax.broadcast((t % 8) * TILE, (16,)).astype(jnp.float32)
        tymin = lax.broadcast((t // 8) * TILE, (16,)).astype(jnp.float32)
        txmax = txmin + (TILE - 1.0)
        tymax = tymin + (TILE - 1.0)

        def step(k, carry):
            sl = pl.ds(k * 16, 16)
            pxv = px_v[sl]; pyv = py_v[sl]
            exv = ex_v[sl]; eyv = ey_v[sl]
            hit = ((pxv - exv <= txmax) & (pxv + exv >= txmin) &
                   (pyv - eyv <= tymax) & (pyv + eyv >= tymin))
            hit_v[sl] = jnp.where(hit, one, zero)
            return carry

        lax.fori_loop(0, N_GAUSS // 16, step, 0)
        pltpu.sync_copy(hit_v, hit_out.at[t])


def _binprep_kernel(hit_ref, pos_ref, nch_ref):
    hf = hit_ref[:, :].astype(jnp.float32)
    ia = lax.broadcasted_iota(jnp.int32, (N_GAUSS, N_GAUSS), 0)
    ja = lax.broadcasted_iota(jnp.int32, (N_GAUSS, N_GAUSS), 1)
    U = (ia < ja).astype(jnp.float32)
    pos = lax.dot(hf, U, preferred_element_type=jnp.float32)
    cnt = pos[:, N_GAUSS - 1:N_GAUSS] + hf[:, N_GAUSS - 1:N_GAUSS]
    nch_ref[:, :] = (cnt.astype(jnp.int32) + (G_CHUNK - 1)) // G_CHUNK
    pos_ref[:, :] = jnp.where(hit_ref[:, :] > 0,
                              pos.astype(jnp.int32), -1)


def _raster_kernel(nch_ref, sorted_ref, pos_ref, img_ref):
    t = pl.program_id(0)
    txmin = (t % 8) * TILE
    tymin = (t // 8) * TILE
    lp = lax.broadcasted_iota(jnp.int32, (1, TPIX), 1)
    gx = (txmin + lp % TILE).astype(jnp.float32)
    gy = (tymin + lp // TILE).astype(jnp.float32)

    ik = lax.broadcasted_iota(jnp.int32, (G_CHUNK, G_CHUNK), 0)
    ij = lax.broadcasted_iota(jnp.int32, (G_CHUNK, G_CHUNK), 1)
    U2 = (ik > ij).astype(jnp.bfloat16)

    pcol = lax.broadcasted_iota(jnp.int32, (G_CHUNK, 1), 0)

    def body(c, state):
        carry, r_acc, g_acc, b_acc = state
        rel = pos_ref[pl.ds(t, 1), :] - c * G_CHUNK
        onehot = (pcol == rel).astype(jnp.float32)
        g = lax.dot(onehot, sorted_ref[:, :],
                    precision=lax.Precision.HIGHEST,
                    preferred_element_type=jnp.float32)
        px = g[:, 1:2]
        py = g[:, 2:3]
        ca = g[:, 3:4]
        cb = g[:, 4:5]
        cc = g[:, 5:6]
        op = g[:, 6:7]
        colr = _bf(g[:, 7:8])
        colg = _bf(g[:, 8:9])
        colb = _bf(g[:, 9:10])

        dx = px - gx
        dy = py - gy
        power = -0.5 * (ca * dx * dx + cc * dy * dy) - cb * dx * dy
        al = op * jnp.exp(jnp.minimum(power, 0.0))
        al = jnp.where(power <= 0.0, jnp.minimum(al, 0.99), 0.0)
        al = jnp.where(al >= ALPHA_MIN, al, 0.0)
        s = jnp.log(1.0 - al)
        s_hi = s.astype(jnp.bfloat16)
        s_lo = (s - s_hi.astype(jnp.float32)).astype(jnp.bfloat16)
        excl = (lax.dot(U2, s_hi, preferred_element_type=jnp.float32)
                + lax.dot(U2, s_lo, preferred_element_type=jnp.float32))
        tprev = jnp.exp(excl + carry)
        w = _bf(al * tprev)
        r_acc = r_acc + jnp.sum(w * colr, axis=0, keepdims=True)
        g_acc = g_acc + jnp.sum(w * colg, axis=0, keepdims=True)
        b_acc = b_acc + jnp.sum(w * colb, axis=0, keepdims=True)
        carry = carry + excl[G_CHUNK - 1:G_CHUNK, :] + s[G_CHUNK - 1:G_CHUNK, :]
        return carry, r_acc, g_acc, b_acc

    zeros = jnp.zeros((1, TPIX), jnp.float32)
    nch = nch_ref[t]
    _, r_acc, g_acc, b_acc = lax.fori_loop(
        0, nch, body, (zeros, zeros, zeros, zeros))
    img_ref[0:1, :] = r_acc
    img_ref[1:2, :] = g_acc
    img_ref[2:3, :] = b_acc


def _make_cull():
    return pl.kernel(
        _cull_kernel,
        mesh=plsc.VectorSubcoreMesh(core_axis_name="c", subcore_axis_name="s"),
        out_type=jax.ShapeDtypeStruct((NTILES, N_GAUSS), jnp.int32),
        scratch_types=[
            pltpu.VMEM((N_GAUSS,), jnp.float32),
            pltpu.VMEM((N_GAUSS,), jnp.float32),
            pltpu.VMEM((N_GAUSS,), jnp.float32),
            pltpu.VMEM((N_GAUSS,), jnp.float32),
            pltpu.VMEM((N_GAUSS,), jnp.int32),
        ],
    )


@jax.jit
def kernel(means3D, means2D, opacities, colors_precomp, scales, rotations,
           backward_mask):
    del means2D, backward_mask
    params, radii = pl.pallas_call(
        _preprocess_kernel,
        out_shape=(
            jax.ShapeDtypeStruct((N_GAUSS, NPARAM), jnp.float32),
            jax.ShapeDtypeStruct((N_GAUSS, 1), jnp.int32),
        ),
    )(means3D, opacities, colors_precomp, scales, rotations)

    params_t = params.T

    sorted_p, sorted_t = pl.pallas_call(
        _sort_kernel,
        out_shape=(
            jax.ShapeDtypeStruct((N_GAUSS, NPARAM), jnp.float32),
            jax.ShapeDtypeStruct((NPARAM, N_GAUSS), jnp.float32),
        ),
    )(params, params_t)

    hit = _make_cull()(sorted_t[1], sorted_t[2], sorted_t[10], sorted_t[11])

    pos, nch = pl.pallas_call(
        _binprep_kernel,
        out_shape=(
            jax.ShapeDtypeStruct((NTILES, N_GAUSS), jnp.int32),
            jax.ShapeDtypeStruct((NTILES, 1), jnp.int32),
        ),
    )(hit)

    img = pl.pallas_call(
        _raster_kernel,
        grid_spec=pltpu.PrefetchScalarGridSpec(
            num_scalar_prefetch=1,
            grid=(NTILES,),
            in_specs=[
                pl.BlockSpec((N_GAUSS, NPARAM), lambda i, *_: (0, 0)),
                pl.BlockSpec((NTILES, N_GAUSS), lambda i, *_: (0, 0)),
            ],
            out_specs=pl.BlockSpec((3, TPIX), lambda i, *_: (0, i)),
        ),
        out_shape=jax.ShapeDtypeStruct((3, NTILES * TPIX), jnp.float32),
    )(nch.reshape(NTILES), sorted_p, pos)

    color = (img.reshape(3, 8, 8, TILE, TILE).transpose(0, 1, 3, 2, 4)
             .reshape(3, H_IMG, W_IMG))
    return color, radii.reshape(N_GAUSS)
